# baseline (speedup 1.0000x reference)
import functools

import jax
import jax.numpy as jnp
from jax import lax
from jax.experimental import pallas as pl
from jax.experimental.pallas import tpu as pltpu
from jax.experimental.pallas import tpu_sc as plsc

B, T, D, MF = 8, 512, 512, 4096
NW = 32
TILES_PER_B = NW // B
CHUNK = 64
NCB = MF // CHUNK
OWN = NCB // TILES_PER_B
L = 16
R = 3
ZR = 32


def _body(xs_hbm, ds_hbm, out_hbm, ds_v, delta_v, idx_v, gbuf, zbuf, shz,
          gsem, wsem, zsem):
    sid = lax.axis_index("s")
    wid = sid * 2 + lax.axis_index("c")
    b = wid // TILES_PER_B
    q = wid % TILES_PER_B
    outbase = b * MF

    ds_copy = pltpu.make_async_copy(ds_hbm.at[pl.ds(b * T, T)], ds_v, zsem)
    ds_copy.start()

    def _zd(i, _):
        for k in range(16):
            delta_v[pl.ds(i * 256 + k * L, L)] = jnp.zeros((L,), jnp.int32)
        return 0
    lax.fori_loop(0, MF // 256, _zd, 0)

    def _zz(r, _):
        for k in range(D // L):
            zbuf[r, pl.ds(k * L, L)] = jnp.zeros((L,), jnp.float32)
        return 0
    lax.fori_loop(0, ZR, _zz, 0)
    ds_copy.wait()

    @pl.when(sid == 0)
    def _():
        pltpu.async_copy(zbuf, shz.at[pl.ds(0, ZR)], zsem)
        pltpu.async_copy(zbuf, shz.at[pl.ds(ZR, ZR)], zsem)
        pltpu.make_async_copy(zbuf, shz.at[pl.ds(0, ZR)], zsem).wait()
        pltpu.make_async_copy(zbuf, shz.at[pl.ds(ZR, ZR)], zsem).wait()
    plsc.subcore_barrier()

    for i in (OWN - 2, OWN - 1):
        row = outbase + (q + i * TILES_PER_B) * CHUNK
        pltpu.async_copy(shz, out_hbm.at[pl.ds(row, CHUNK)], zsem)

    ones = jnp.ones((L,), jnp.int32)

    def _p1(j, tot):
        d = ds_v[pl.ds(j * L, L)]
        inc = plsc.cumsum(d)
        a = inc - d + tot
        m = a < MF
        plsc.addupdate_scatter(delta_v, [jnp.clip(a, 0, MF - 1)], ones,
                               mask=m)
        return tot + jnp.sum(d)

    total = lax.fori_loop(0, T // L, _p1, jnp.int32(0))

    k_valid = jnp.clip((total - q * CHUNK + (TILES_PER_B * CHUNK - 1))
                       // (TILES_PER_B * CHUNK), 0, OWN)

    def _zw(i, _):
        c = q + i * TILES_PER_B
        row = outbase + c * CHUNK
        pltpu.async_copy(shz, out_hbm.at[pl.ds(row, CHUNK)], zsem)
        return 0
    lax.fori_loop(k_valid, OWN - 2, _zw, 0)

    nscan = jnp.clip((total + CHUNK - 1) // CHUNK, 0, NCB)

    def _scan(c, cnt):
        for j in range(CHUNK // L):
            dl = delta_v[pl.ds(c * CHUNK + j * L, L)]
            pos = plsc.cumsum(dl) + cnt
            idx_v[c, pl.ds(j * L, L)] = jnp.clip(pos - 1, 0, T - 1) + b * T
            cnt = cnt + jnp.sum(dl)
        return cnt

    cnt8 = lax.fori_loop(0, jnp.minimum(nscan, 8), _scan, jnp.int32(0))

    @pl.when(k_valid > 0)
    def _():
        pltpu.async_copy(xs_hbm.at[idx_v.at[q]], gbuf.at[0], gsem.at[0])

    @pl.when(k_valid > 1)
    def _():
        pltpu.async_copy(xs_hbm.at[idx_v.at[q + TILES_PER_B]], gbuf.at[1],
                         gsem.at[1])

    lax.fori_loop(8, nscan, _scan, cnt8)

    def _pipe(i, _):
        c = q + i * TILES_PER_B
        p = lax.rem(i, R)
        pltpu.make_async_copy(xs_hbm.at[idx_v.at[c]], gbuf.at[p],
                              gsem.at[p]).wait()

        nv = jnp.clip(total - c * CHUNK, 0, CHUNK)

        @pl.when(nv < CHUNK)
        def _():
            def _zr(r, _):
                for k in range(D // L):
                    gbuf[p, r, pl.ds(k * L, L)] = jnp.zeros((L,), jnp.float32)
                return 0
            lax.fori_loop(nv, CHUNK, _zr, 0)

        pltpu.async_copy(gbuf.at[p],
                         out_hbm.at[pl.ds(outbase + c * CHUNK, CHUNK)],
                         wsem.at[p])

        @pl.when(i + 2 < k_valid)
        def _():
            p2 = lax.rem(i + 2, R)

            @pl.when(i >= 1)
            def _():
                pltpu.make_async_copy(
                    gbuf.at[p2], out_hbm.at[pl.ds(outbase, CHUNK)],
                    wsem.at[p2]).wait()
            pltpu.async_copy(xs_hbm.at[idx_v.at[c + 2 * TILES_PER_B]],
                             gbuf.at[p2], gsem.at[p2])
        return 0

    lax.fori_loop(0, k_valid, _pipe, 0)

    @pl.when(k_valid >= 3)
    def _():
        p = lax.rem(k_valid, R)
        pltpu.make_async_copy(gbuf.at[p], out_hbm.at[pl.ds(outbase, CHUNK)],
                              wsem.at[p]).wait()

    @pl.when(k_valid >= 2)
    def _():
        p = lax.rem(k_valid + 1, R)
        pltpu.make_async_copy(gbuf.at[p], out_hbm.at[pl.ds(outbase, CHUNK)],
                              wsem.at[p]).wait()

    @pl.when(k_valid >= 1)
    def _():
        p = lax.rem(k_valid + 2, R)
        pltpu.make_async_copy(gbuf.at[p], out_hbm.at[pl.ds(outbase, CHUNK)],
                              wsem.at[p]).wait()

    def _zdrain(i, _):
        pltpu.make_async_copy(shz, out_hbm.at[pl.ds(outbase, CHUNK)],
                              zsem).wait()
        return 0
    lax.fori_loop(k_valid, OWN, _zdrain, 0)


_mesh = plsc.VectorSubcoreMesh(core_axis_name="c", subcore_axis_name="s")

_regulate = functools.partial(
    pl.kernel,
    out_type=jax.ShapeDtypeStruct((B * MF, D), jnp.float32),
    mesh=_mesh,
    compiler_params=pltpu.CompilerParams(needs_layout_passes=False),
    scratch_types=[
        pltpu.VMEM((T,), jnp.int32),
        pltpu.VMEM((MF,), jnp.int32),
        pltpu.VMEM((NCB, CHUNK), jnp.int32),
        pltpu.VMEM((R, CHUNK, D), jnp.float32),
        pltpu.VMEM((ZR, D), jnp.float32),
        pltpu.VMEM_SHARED((CHUNK, D), jnp.float32),
        pltpu.SemaphoreType.DMA((R,)),
        pltpu.SemaphoreType.DMA((R,)),
        pltpu.SemaphoreType.DMA,
    ],
)(_body)


def kernel(xs, ds, max_frame):
    del max_frame
    out = _regulate(xs.reshape(B * T, D), ds.reshape(B * T))
    return out.reshape(B, MF, D)


# speedup vs baseline: 1.0189x; 1.0189x over previous
import functools

import jax
import jax.numpy as jnp
from jax import lax
from jax.experimental import pallas as pl
from jax.experimental.pallas import tpu as pltpu
from jax.experimental.pallas import tpu_sc as plsc

B, T, D, MF = 8, 512, 512, 4096
NW = 32
TILES_PER_B = NW // B
CHUNK = 64
NCB = MF // CHUNK
OWN = NCB // TILES_PER_B
L = 16
R = 3
ZR = 32


def _body(xs_hbm, ds_hbm, out_hbm, ds_v, delta_v, idx_v, gbuf, zbuf, shz,
          gsem, wsem, zsem, ssem):
    sid = lax.axis_index("s")
    wid = sid * 2 + lax.axis_index("c")
    b = wid // TILES_PER_B
    q = wid % TILES_PER_B
    outbase = b * MF

    ds_copy = pltpu.make_async_copy(ds_hbm.at[pl.ds(b * T, T)], ds_v, zsem)
    ds_copy.start()

    def _zd(i, _):
        for k in range(16):
            delta_v[pl.ds(i * 256 + k * L, L)] = jnp.zeros((L,), jnp.int32)
        return 0
    lax.fori_loop(0, MF // 256, _zd, 0)

    def _zz(r, _):
        for k in range(D // L):
            zbuf[r, pl.ds(k * L, L)] = jnp.zeros((L,), jnp.float32)
        return 0
    lax.fori_loop(0, ZR, _zz, 0)
    ds_copy.wait()

    @pl.when(sid == 0)
    def _():
        pltpu.async_copy(zbuf, shz.at[pl.ds(0, ZR)], ssem)
        pltpu.async_copy(zbuf, shz.at[pl.ds(ZR, ZR)], ssem)

    for i in (OWN - 2, OWN - 1):
        row = outbase + (q + i * TILES_PER_B) * CHUNK
        pltpu.async_copy(zbuf, out_hbm.at[pl.ds(row, ZR)], zsem)
        pltpu.async_copy(zbuf, out_hbm.at[pl.ds(row + ZR, ZR)], zsem)

    ones = jnp.ones((L,), jnp.int32)

    def _p1(j, tot):
        d = ds_v[pl.ds(j * L, L)]
        inc = plsc.cumsum(d)
        a = inc - d + tot
        m = a < MF
        plsc.addupdate_scatter(delta_v, [jnp.clip(a, 0, MF - 1)], ones,
                               mask=m)
        return tot + jnp.sum(d)

    total = lax.fori_loop(0, T // L, _p1, jnp.int32(0))

    @pl.when(sid == 0)
    def _():
        pltpu.make_async_copy(zbuf, shz.at[pl.ds(0, ZR)], ssem).wait()
        pltpu.make_async_copy(zbuf, shz.at[pl.ds(ZR, ZR)], ssem).wait()
    plsc.subcore_barrier()

    k_valid = jnp.clip((total - q * CHUNK + (TILES_PER_B * CHUNK - 1))
                       // (TILES_PER_B * CHUNK), 0, OWN)

    def _zw(i, _):
        c = q + i * TILES_PER_B
        row = outbase + c * CHUNK
        pltpu.async_copy(shz, out_hbm.at[pl.ds(row, CHUNK)], zsem)
        return 0
    lax.fori_loop(k_valid, OWN - 2, _zw, 0)

    nscan = jnp.clip((total + CHUNK - 1) // CHUNK, 0, NCB)

    def _scan(c, cnt):
        for j in range(CHUNK // L):
            dl = delta_v[pl.ds(c * CHUNK + j * L, L)]
            pos = plsc.cumsum(dl) + cnt
            idx_v[c, pl.ds(j * L, L)] = jnp.clip(pos - 1, 0, T - 1) + b * T
            cnt = cnt + jnp.sum(dl)
        return cnt

    cnt8 = lax.fori_loop(0, jnp.minimum(nscan, 8), _scan, jnp.int32(0))

    @pl.when(k_valid > 0)
    def _():
        pltpu.async_copy(xs_hbm.at[idx_v.at[q]], gbuf.at[0], gsem.at[0])

    @pl.when(k_valid > 1)
    def _():
        pltpu.async_copy(xs_hbm.at[idx_v.at[q + TILES_PER_B]], gbuf.at[1],
                         gsem.at[1])

    lax.fori_loop(8, nscan, _scan, cnt8)

    def _pipe(i, _):
        c = q + i * TILES_PER_B
        p = lax.rem(i, R)
        pltpu.make_async_copy(xs_hbm.at[idx_v.at[c]], gbuf.at[p],
                              gsem.at[p]).wait()

        nv = jnp.clip(total - c * CHUNK, 0, CHUNK)

        @pl.when(nv < CHUNK)
        def _():
            def _zr(r, _):
                for k in range(D // L):
                    gbuf[p, r, pl.ds(k * L, L)] = jnp.zeros((L,), jnp.float32)
                return 0
            lax.fori_loop(nv, CHUNK, _zr, 0)

        pltpu.async_copy(gbuf.at[p],
                         out_hbm.at[pl.ds(outbase + c * CHUNK, CHUNK)],
                         wsem.at[p])

        @pl.when(i + 2 < k_valid)
        def _():
            p2 = lax.rem(i + 2, R)

            @pl.when(i >= 1)
            def _():
                pltpu.make_async_copy(
                    gbuf.at[p2], out_hbm.at[pl.ds(outbase, CHUNK)],
                    wsem.at[p2]).wait()
            pltpu.async_copy(xs_hbm.at[idx_v.at[c + 2 * TILES_PER_B]],
                             gbuf.at[p2], gsem.at[p2])
        return 0

    lax.fori_loop(0, k_valid, _pipe, 0)

    @pl.when(k_valid >= 3)
    def _():
        p = lax.rem(k_valid, R)
        pltpu.make_async_copy(gbuf.at[p], out_hbm.at[pl.ds(outbase, CHUNK)],
                              wsem.at[p]).wait()

    @pl.when(k_valid >= 2)
    def _():
        p = lax.rem(k_valid + 1, R)
        pltpu.make_async_copy(gbuf.at[p], out_hbm.at[pl.ds(outbase, CHUNK)],
                              wsem.at[p]).wait()

    @pl.when(k_valid >= 1)
    def _():
        p = lax.rem(k_valid + 2, R)
        pltpu.make_async_copy(gbuf.at[p], out_hbm.at[pl.ds(outbase, CHUNK)],
                              wsem.at[p]).wait()

    def _zdrain(i, _):
        pltpu.make_async_copy(shz, out_hbm.at[pl.ds(outbase, CHUNK)],
                              zsem).wait()
        return 0
    lax.fori_loop(k_valid, OWN - 2, _zdrain, 0)
    for _ in range(4):
        pltpu.make_async_copy(zbuf, out_hbm.at[pl.ds(outbase, ZR)],
                              zsem).wait()


_mesh = plsc.VectorSubcoreMesh(core_axis_name="c", subcore_axis_name="s")

_regulate = functools.partial(
    pl.kernel,
    out_type=jax.ShapeDtypeStruct((B * MF, D), jnp.float32),
    mesh=_mesh,
    compiler_params=pltpu.CompilerParams(needs_layout_passes=False),
    scratch_types=[
        pltpu.VMEM((T,), jnp.int32),
        pltpu.VMEM((MF,), jnp.int32),
        pltpu.VMEM((NCB, CHUNK), jnp.int32),
        pltpu.VMEM((R, CHUNK, D), jnp.float32),
        pltpu.VMEM((ZR, D), jnp.float32),
        pltpu.VMEM_SHARED((CHUNK, D), jnp.float32),
        pltpu.SemaphoreType.DMA((R,)),
        pltpu.SemaphoreType.DMA((R,)),
        pltpu.SemaphoreType.DMA,
        pltpu.SemaphoreType.DMA,
    ],
)(_body)


def kernel(xs, ds, max_frame):
    del max_frame
    out = _regulate(xs.reshape(B * T, D), ds.reshape(B * T))
    return out.reshape(B, MF, D)


# speedup vs baseline: 1.0204x; 1.0015x over previous
import functools

import jax
import jax.numpy as jnp
from jax import lax
from jax.experimental import pallas as pl
from jax.experimental.pallas import tpu as pltpu
from jax.experimental.pallas import tpu_sc as plsc

B, T, D, MF = 8, 512, 512, 4096
NW = 32
TILES_PER_B = NW // B
CHUNK = 64
NCB = MF // CHUNK
OWN = NCB // TILES_PER_B
L = 16
R = 3
ZR = 32


def _body(xs_hbm, ds_hbm, out_hbm, ds_v, delta_v, idx_v, gbuf, zbuf, shz,
          gsem, wsem, zsem, ssem):
    sid = lax.axis_index("s")
    wid = sid * 2 + lax.axis_index("c")
    b = wid // TILES_PER_B
    q = wid % TILES_PER_B
    outbase = b * MF

    ds_copy = pltpu.make_async_copy(ds_hbm.at[pl.ds(b * T, T)], ds_v, zsem)
    ds_copy.start()

    def _zd(i, _):
        for k in range(16):
            delta_v[pl.ds(i * 256 + k * L, L)] = jnp.zeros((L,), jnp.int32)
        return 0
    lax.fori_loop(0, MF // 256, _zd, 0)

    def _zz(r, _):
        for k in range(D // L):
            zbuf[r, pl.ds(k * L, L)] = jnp.zeros((L,), jnp.float32)
        return 0
    lax.fori_loop(0, ZR, _zz, 0)
    ds_copy.wait()

    @pl.when(sid == 0)
    def _():
        pltpu.async_copy(zbuf, shz.at[pl.ds(0, ZR)], ssem)
        pltpu.async_copy(zbuf, shz.at[pl.ds(ZR, ZR)], ssem)

    for i in (OWN - 2, OWN - 1):
        row = outbase + (q + i * TILES_PER_B) * CHUNK
        pltpu.async_copy(zbuf, out_hbm.at[pl.ds(row, ZR)], zsem)
        pltpu.async_copy(zbuf, out_hbm.at[pl.ds(row + ZR, ZR)], zsem)

    ones = jnp.ones((L,), jnp.int32)

    def _p1(j, tot):
        d = ds_v[pl.ds(j * L, L)]
        inc = plsc.cumsum(d)
        a = inc - d + tot
        m = a < MF
        plsc.addupdate_scatter(delta_v, [jnp.clip(a, 0, MF - 1)], ones,
                               mask=m)
        return tot + inc[L - 1]

    total = lax.fori_loop(0, T // L, _p1, jnp.int32(0))

    @pl.when(sid == 0)
    def _():
        pltpu.make_async_copy(zbuf, shz.at[pl.ds(0, ZR)], ssem).wait()
        pltpu.make_async_copy(zbuf, shz.at[pl.ds(ZR, ZR)], ssem).wait()
    plsc.subcore_barrier()

    k_valid = jnp.clip((total - q * CHUNK + (TILES_PER_B * CHUNK - 1))
                       // (TILES_PER_B * CHUNK), 0, OWN)

    def _zw(i, _):
        c = q + i * TILES_PER_B
        row = outbase + c * CHUNK
        pltpu.async_copy(shz, out_hbm.at[pl.ds(row, CHUNK)], zsem)
        return 0
    lax.fori_loop(k_valid, OWN - 2, _zw, 0)

    nscan = jnp.clip((total + CHUNK - 1) // CHUNK, 0, NCB)

    def _scan(c, cnt):
        for j in range(CHUNK // L):
            dl = delta_v[pl.ds(c * CHUNK + j * L, L)]
            pos = plsc.cumsum(dl) + cnt
            idx_v[c, pl.ds(j * L, L)] = jnp.clip(pos - 1, 0, T - 1) + b * T
            cnt = pos[L - 1]
        return cnt

    cnt8 = lax.fori_loop(0, jnp.minimum(nscan, 8), _scan, jnp.int32(0))

    @pl.when(k_valid > 0)
    def _():
        pltpu.async_copy(xs_hbm.at[idx_v.at[q]], gbuf.at[0], gsem.at[0])

    @pl.when(k_valid > 1)
    def _():
        pltpu.async_copy(xs_hbm.at[idx_v.at[q + TILES_PER_B]], gbuf.at[1],
                         gsem.at[1])

    lax.fori_loop(8, nscan, _scan, cnt8)

    def _pipe(i, _):
        c = q + i * TILES_PER_B
        p = lax.rem(i, R)
        pltpu.make_async_copy(xs_hbm.at[idx_v.at[c]], gbuf.at[p],
                              gsem.at[p]).wait()

        nv = jnp.clip(total - c * CHUNK, 0, CHUNK)

        @pl.when(nv < CHUNK)
        def _():
            def _zr(r, _):
                for k in range(D // L):
                    gbuf[p, r, pl.ds(k * L, L)] = jnp.zeros((L,), jnp.float32)
                return 0
            lax.fori_loop(nv, CHUNK, _zr, 0)

        pltpu.async_copy(gbuf.at[p],
                         out_hbm.at[pl.ds(outbase + c * CHUNK, CHUNK)],
                         wsem.at[p])

        @pl.when(i + 2 < k_valid)
        def _():
            p2 = lax.rem(i + 2, R)

            @pl.when(i >= 1)
            def _():
                pltpu.make_async_copy(
                    gbuf.at[p2], out_hbm.at[pl.ds(outbase, CHUNK)],
                    wsem.at[p2]).wait()
            pltpu.async_copy(xs_hbm.at[idx_v.at[c + 2 * TILES_PER_B]],
                             gbuf.at[p2], gsem.at[p2])
        return 0

    lax.fori_loop(0, k_valid, _pipe, 0)

    @pl.when(k_valid >= 3)
    def _():
        p = lax.rem(k_valid, R)
        pltpu.make_async_copy(gbuf.at[p], out_hbm.at[pl.ds(outbase, CHUNK)],
                              wsem.at[p]).wait()

    @pl.when(k_valid >= 2)
    def _():
        p = lax.rem(k_valid + 1, R)
        pltpu.make_async_copy(gbuf.at[p], out_hbm.at[pl.ds(outbase, CHUNK)],
                              wsem.at[p]).wait()

    @pl.when(k_valid >= 1)
    def _():
        p = lax.rem(k_valid + 2, R)
        pltpu.make_async_copy(gbuf.at[p], out_hbm.at[pl.ds(outbase, CHUNK)],
                              wsem.at[p]).wait()

    def _zdrain(i, _):
        pltpu.make_async_copy(shz, out_hbm.at[pl.ds(outbase, CHUNK)],
                              zsem).wait()
        return 0
    lax.fori_loop(k_valid, OWN - 2, _zdrain, 0)
    for _ in range(4):
        pltpu.make_async_copy(zbuf, out_hbm.at[pl.ds(outbase, ZR)],
                              zsem).wait()


_mesh = plsc.VectorSubcoreMesh(core_axis_name="c", subcore_axis_name="s")

_regulate = functools.partial(
    pl.kernel,
    out_type=jax.ShapeDtypeStruct((B * MF, D), jnp.float32),
    mesh=_mesh,
    compiler_params=pltpu.CompilerParams(needs_layout_passes=False),
    scratch_types=[
        pltpu.VMEM((T,), jnp.int32),
        pltpu.VMEM((MF,), jnp.int32),
        pltpu.VMEM((NCB, CHUNK), jnp.int32),
        pltpu.VMEM((R, CHUNK, D), jnp.float32),
        pltpu.VMEM((ZR, D), jnp.float32),
        pltpu.VMEM_SHARED((CHUNK, D), jnp.float32),
        pltpu.SemaphoreType.DMA((R,)),
        pltpu.SemaphoreType.DMA((R,)),
        pltpu.SemaphoreType.DMA,
        pltpu.SemaphoreType.DMA,
    ],
)(_body)


def kernel(xs, ds, max_frame):
    del max_frame
    out = _regulate(xs.reshape(B * T, D), ds.reshape(B * T))
    return out.reshape(B, MF, D)
